# Initial kernel scaffold; baseline (speedup 1.0000x reference)
import functools

import jax
import jax.numpy as jnp
from jax import lax
from jax.experimental import pallas as pl
from jax.experimental.pallas import tpu as pltpu
from jax.experimental.pallas import tpu_sc as plsc

_N = 10000
_E = 320000
_H = 64

_NC = 2
_NS = 16
_NW = _NC * _NS
_PW = _E // _NW
_C = 80
_NCH = _PW // _C
_K = 5
_NRND = _NCH // _K
_NPS = _N // _NS

_MESH = plsc.VectorSubcoreMesh(core_axis_name="c", subcore_axis_name="s")


@functools.partial(
    pl.kernel,
    out_type=[
        jax.ShapeDtypeStruct((_E, _H), jnp.float32),
        jax.ShapeDtypeStruct((_E, _H), jnp.float32),
    ],
    mesh=_MESH,
    scratch_types=[
        pltpu.VMEM((_NCH, _C), jnp.int32),
        pltpu.VMEM((_NCH, _C), jnp.int32),
        pltpu.VMEM((2 * _K, _C, _H), jnp.float32),
        pltpu.SemaphoreType.DMA,
        pltpu.SemaphoreType.DMA,
    ],
)
def _sc_gather(ps_hbm, pd_hbm, src2_hbm, dst2_hbm, gs_out, gd_out,
               sidx, didx, rows, gsem, ssem):
    wid = lax.axis_index("s") * _NC + lax.axis_index("c")
    base = wid * _PW
    ibase = wid * _NCH
    pltpu.sync_copy(src2_hbm.at[pl.ds(ibase, _NCH)], sidx)
    pltpu.sync_copy(dst2_hbm.at[pl.ds(ibase, _NCH)], didx)

    def round_body(t, carry):
        j0 = t * _K
        fires = []
        for u in range(_K):
            fires.append(pltpu.async_copy(
                ps_hbm.at[sidx.at[j0 + u]], rows.at[2 * u], gsem))
            fires.append(pltpu.async_copy(
                pd_hbm.at[didx.at[j0 + u]], rows.at[2 * u + 1], gsem))
        for h in fires:
            h.wait()
        drains = []
        for u in range(_K):
            off = base + (j0 + u) * _C
            drains.append(pltpu.async_copy(
                rows.at[2 * u], gs_out.at[pl.ds(off, _C)], ssem))
            drains.append(pltpu.async_copy(
                rows.at[2 * u + 1], gd_out.at[pl.ds(off, _C)], ssem))
        for h in drains:
            h.wait()
        return carry

    lax.fori_loop(0, _NRND, round_body, 0)


@functools.partial(
    pl.kernel,
    out_type=jax.ShapeDtypeStruct((_NC, _N, _H), jnp.float32),
    mesh=_MESH,
    scratch_types=[
        pltpu.VMEM((_NCH, _C), jnp.int32),
        pltpu.VMEM((_K, _C, _H), jnp.float32),
        pltpu.VMEM_SHARED((_N, _H), jnp.float32),
        pltpu.SemaphoreType.DMA,
        pltpu.SemaphoreType.DMA,
    ],
)
def _sc_scatter(enew_hbm, dst2_hbm, zeros_hbm, out_hbm,
                didx, rows, acc, lsem, ssem):
    c = lax.axis_index("c")
    s = lax.axis_index("s")
    wid = s * _NC + c
    base = wid * _PW
    pltpu.sync_copy(zeros_hbm, acc.at[pl.ds(s * _NPS, _NPS)])
    pltpu.sync_copy(dst2_hbm.at[pl.ds(wid * _NCH, _NCH)], didx)
    plsc.subcore_barrier()

    def round_body(t, carry):
        j0 = t * _K
        fires = []
        for u in range(_K):
            off = base + (j0 + u) * _C
            fires.append(pltpu.async_copy(
                enew_hbm.at[pl.ds(off, _C)], rows.at[u], lsem))
        for h in fires:
            h.wait()
        adds = []
        for u in range(_K):
            adds.append(pltpu.async_copy(
                rows.at[u], acc.at[didx.at[j0 + u]], ssem, add=True))
        for h in adds:
            h.wait()
        return carry

    lax.fori_loop(0, _NRND, round_body, 0)
    plsc.subcore_barrier()
    pltpu.sync_copy(acc.at[pl.ds(s * _NPS, _NPS)],
                    out_hbm.at[c, pl.ds(s * _NPS, _NPS)])


_BN = 1000
_BE = 2000


def _dot(a, b):
    return jnp.dot(a, b, preferred_element_type=jnp.float32)


def _full(shape):
    return pl.BlockSpec(shape, lambda i: (0,) * len(shape))


def _node_enc_body(x_ref, w1, b1, w2, b2, w1s, w1d, hn_ref, ps_ref, pd_ref):
    h = jnp.maximum(_dot(x_ref[...], w1[...]) + b1[...], 0.0)
    hn = _dot(h, w2[...]) + b2[...]
    hn_ref[...] = hn
    ps_ref[...] = _dot(hn, w1s[...])
    pd_ref[...] = _dot(hn, w1d[...])


def _edge_enc_body(ea_ref, w1, b1, w2, b2, he_ref):
    h = jnp.maximum(_dot(ea_ref[...], w1[...]) + b1[...], 0.0)
    he_ref[...] = _dot(h, w2[...]) + b2[...]


def _edge_step_body(he_ref, gs_ref, gd_ref, w1e, b1, w2, b2, en_ref, hen_ref):
    z = jnp.maximum(
        _dot(he_ref[...], w1e[...]) + gs_ref[...] + gd_ref[...] + b1[...], 0.0)
    en = _dot(z, w2[...]) + b2[...]
    en_ref[...] = en
    hen_ref[...] = he_ref[...] + en


def _edge_last_body(he_ref, gs_ref, gd_ref, w1e, b1, w2, b2, en_ref):
    z = jnp.maximum(
        _dot(he_ref[...], w1e[...]) + gs_ref[...] + gd_ref[...] + b1[...], 0.0)
    en_ref[...] = _dot(z, w2[...]) + b2[...]


def _node_upd_body(hn_ref, parts_ref, w1h, w1a, b1, w2, b2, w1s, w1d,
                   hn2_ref, ps_ref, pd_ref):
    agg = parts_ref[0] + parts_ref[1]
    h = jnp.maximum(
        _dot(hn_ref[...], w1h[...]) + _dot(agg, w1a[...]) + b1[...], 0.0)
    hn2 = hn_ref[...] + _dot(h, w2[...]) + b2[...]
    hn2_ref[...] = hn2
    ps_ref[...] = _dot(hn2, w1s[...])
    pd_ref[...] = _dot(hn2, w1d[...])


def _node_final_body(hn_ref, parts_ref, w1h, w1a, b1, w2, b2, dw1, db1,
                     dw2, db2, out_ref):
    agg = parts_ref[0] + parts_ref[1]
    h = jnp.maximum(
        _dot(hn_ref[...], w1h[...]) + _dot(agg, w1a[...]) + b1[...], 0.0)
    hn2 = hn_ref[...] + _dot(h, w2[...]) + b2[...]
    d = jnp.maximum(_dot(hn2, dw1[...]) + db1[...], 0.0)
    out_ref[...] = _dot(d, dw2[...]) + db2[...]


def kernel(x, edge_index, edge_attr, params):
    f32 = jnp.float32
    src = edge_index[0].astype(jnp.int32)
    dst = edge_index[1].astype(jnp.int32)
    src2 = src.reshape(_E // _C, _C)
    dst2 = dst.reshape(_E // _C, _C)
    zeros_tile = jnp.zeros((_NPS, _H), f32)

    pn, pe = params["enc_node"], params["enc_edge"]
    dec = params["dec_node"]

    def row(b):
        return b.reshape(1, -1)

    ew = [params["proc"][i]["edge"] for i in range(2)]
    nw = [params["proc"][i]["node"] for i in range(2)]

    grid_n = _N // _BN
    grid_e = _E // _BE
    bspec_n = pl.BlockSpec((_BN, _H), lambda i: (i, 0))
    bspec_e = pl.BlockSpec((_BE, _H), lambda i: (i, 0))
    w_spec = _full((_H, _H))
    b_spec = _full((1, _H))

    hn, ps1, pd1 = pl.pallas_call(
        _node_enc_body,
        grid=(grid_n,),
        in_specs=[pl.BlockSpec((_BN, 128), lambda i: (i, 0)),
                  _full((128, _H)), b_spec, w_spec, b_spec, w_spec, w_spec],
        out_specs=[bspec_n, bspec_n, bspec_n],
        out_shape=[jax.ShapeDtypeStruct((_N, _H), f32)] * 3,
    )(x, pn["w1"], row(pn["b1"]), pn["w2"], row(pn["b2"]),
      ew[0]["w1"][64:128], ew[0]["w1"][128:192])

    he = pl.pallas_call(
        _edge_enc_body,
        grid=(grid_e,),
        in_specs=[pl.BlockSpec((_BE, 16), lambda i: (i, 0)),
                  _full((16, _H)), b_spec, w_spec, b_spec],
        out_specs=bspec_e,
        out_shape=jax.ShapeDtypeStruct((_E, _H), f32),
    )(edge_attr, pe["w1"], row(pe["b1"]), pe["w2"], row(pe["b2"]))

    for step in range(2):
        gs, gd = _sc_gather(ps1, pd1, src2, dst2)
        if step == 0:
            en, he = pl.pallas_call(
                _edge_step_body,
                grid=(grid_e,),
                in_specs=[bspec_e, bspec_e, bspec_e,
                          w_spec, b_spec, w_spec, b_spec],
                out_specs=[bspec_e, bspec_e],
                out_shape=[jax.ShapeDtypeStruct((_E, _H), f32)] * 2,
            )(he, gs, gd, ew[step]["w1"][0:64], row(ew[step]["b1"]),
              ew[step]["w2"], row(ew[step]["b2"]))
        else:
            en = pl.pallas_call(
                _edge_last_body,
                grid=(grid_e,),
                in_specs=[bspec_e, bspec_e, bspec_e,
                          w_spec, b_spec, w_spec, b_spec],
                out_specs=bspec_e,
                out_shape=jax.ShapeDtypeStruct((_E, _H), f32),
            )(he, gs, gd, ew[step]["w1"][0:64], row(ew[step]["b1"]),
              ew[step]["w2"], row(ew[step]["b2"]))

        parts = _sc_scatter(en, dst2, zeros_tile)

        parts_spec = pl.BlockSpec((_NC, _BN, _H), lambda i: (0, i, 0))
        if step == 0:
            hn, ps1, pd1 = pl.pallas_call(
                _node_upd_body,
                grid=(grid_n,),
                in_specs=[bspec_n, parts_spec, w_spec, w_spec, b_spec,
                          w_spec, b_spec, w_spec, w_spec],
                out_specs=[bspec_n, bspec_n, bspec_n],
                out_shape=[jax.ShapeDtypeStruct((_N, _H), f32)] * 3,
            )(hn, parts, nw[step]["w1"][0:64], nw[step]["w1"][64:128],
              row(nw[step]["b1"]), nw[step]["w2"], row(nw[step]["b2"]),
              ew[1]["w1"][64:128], ew[1]["w1"][128:192])
        else:
            out = pl.pallas_call(
                _node_final_body,
                grid=(grid_n,),
                in_specs=[bspec_n, parts_spec, w_spec, w_spec, b_spec,
                          w_spec, b_spec, w_spec, b_spec,
                          _full((_H, 4)), _full((1, 4))],
                out_specs=pl.BlockSpec((_BN, 4), lambda i: (i, 0)),
                out_shape=jax.ShapeDtypeStruct((_N, 4), f32),
            )(hn, parts, nw[step]["w1"][0:64], nw[step]["w1"][64:128],
              row(nw[step]["b1"]), nw[step]["w2"], row(nw[step]["b2"]),
              dec["w1"], row(dec["b1"]), dec["w2"], row(dec["b2"]))
    return out


# speedup vs baseline: 2.4428x; 2.4428x over previous
import functools

import jax
import jax.numpy as jnp
from jax import lax
from jax.experimental import pallas as pl
from jax.experimental.pallas import tpu as pltpu
from jax.experimental.pallas import tpu_sc as plsc

_N = 10000
_E = 320000
_H = 64

_NC = 2
_NS = 16
_NW = _NC * _NS
_PW = _E // _NW
_C = 80
_NCH = _PW // _C
_K = 5
_NRND = _NCH // _K
_HN = _N // 2
_NWB = 1000

_MESH = plsc.VectorSubcoreMesh(core_axis_name="c", subcore_axis_name="s")


@functools.partial(
    pl.kernel,
    out_type=[
        jax.ShapeDtypeStruct((_E, 2 * _H), jnp.float32),
        jax.ShapeDtypeStruct((_E, 2 * _H), jnp.float32),
    ],
    mesh=_MESH,
    scratch_types=[
        pltpu.VMEM((_NCH, _C), jnp.int32),
        pltpu.VMEM((_NCH, _C), jnp.int32),
        pltpu.VMEM((_K, _C, 2 * _H), jnp.float32),
        pltpu.SemaphoreType.DMA,
        pltpu.SemaphoreType.DMA,
    ],
)
def _sc_gather(pt_hbm, src2_hbm, dst2_hbm, gs_out, gd_out,
               sidx, didx, rows, gsem, ssem):
    wid = lax.axis_index("s") * _NC + lax.axis_index("c")
    base = wid * _PW
    pltpu.sync_copy(src2_hbm.at[wid], sidx)
    pltpu.sync_copy(dst2_hbm.at[wid], didx)

    def make_round(idx_ref, out_ref):
        def round_body(t, carry):
            j0 = t * _K
            fires = []
            for u in range(_K):
                fires.append(pltpu.async_copy(
                    pt_hbm.at[idx_ref.at[j0 + u]], rows.at[u], gsem))
            for h in fires:
                h.wait()
            drains = []
            for u in range(_K):
                off = base + (j0 + u) * _C
                drains.append(pltpu.async_copy(
                    rows.at[u], out_ref.at[pl.ds(off, _C)], ssem))
            for h in drains:
                h.wait()
            return carry
        return round_body

    lax.fori_loop(0, _NRND, make_round(sidx, gs_out), 0)
    lax.fori_loop(0, _NRND, make_round(didx, gd_out), 0)


@functools.partial(
    pl.kernel,
    out_type=jax.ShapeDtypeStruct((_NC, _N, 2 * _H), jnp.float32),
    mesh=_MESH,
    scratch_types=[
        pltpu.VMEM((_NCH, _C), jnp.int32),
        pltpu.VMEM((_NCH, _C), jnp.int32),
        pltpu.VMEM((_K, _C, 2 * _H), jnp.float32),
        pltpu.VMEM_SHARED((_HN + 16, 2 * _H), jnp.float32),
        pltpu.SemaphoreType.DMA,
    ],
)
def _sc_scatter(en3_hbm, dst2_hbm, zeros_hbm, out_hbm,
                didx, didx2, rows, acc, lsem):
    c = lax.axis_index("c")
    s = lax.axis_index("s")
    wid = s * _NC + c
    pltpu.sync_copy(dst2_hbm.at[wid], didx)

    for p in range(2):
        lo = p * _HN

        @pl.when(s < _HN // _NWB)
        def _init():
            pltpu.sync_copy(zeros_hbm, acc.at[pl.ds(s * _NWB, _NWB)])

        @pl.when(s == _HN // _NWB)
        def _init_dump():
            pltpu.sync_copy(zeros_hbm.at[pl.ds(0, 16)],
                            acc.at[pl.ds(_HN, 16)])

        def remap_body(j, carry):
            for v in range(_C // 16):
                idx = didx[j, pl.ds(v * 16, 16)]
                ok = (idx >= lo) & (idx < lo + _HN)
                didx2[j, pl.ds(v * 16, 16)] = jnp.where(ok, idx - lo, _HN)
            return carry

        lax.fori_loop(0, _NCH, remap_body, 0)
        plsc.subcore_barrier()

        def round_body(t, carry):
            j0 = t * _K
            fires = []
            for u in range(_K):
                fires.append(pltpu.async_copy(
                    en3_hbm.at[wid * _NCH + j0 + u], rows.at[u], lsem))
            for h in fires:
                h.wait()
            for u in range(_K):
                pltpu.sync_copy(rows.at[u], acc.at[didx2.at[j0 + u]],
                                add=True)
            return carry

        lax.fori_loop(0, _NRND, round_body, 0)
        plsc.subcore_barrier()

        @pl.when(s < _HN // _NWB)
        def _writeback():
            pltpu.sync_copy(acc.at[pl.ds(s * _NWB, _NWB)],
                            out_hbm.at[c, pl.ds(lo + s * _NWB, _NWB)])

        plsc.subcore_barrier()


_BN = 1000
_BE = 2000


def _dot(a, b):
    return jnp.dot(a, b, preferred_element_type=jnp.float32)


def _full(shape):
    return pl.BlockSpec(shape, lambda i: (0,) * len(shape))


def _node_enc_body(x_ref, w1, b1, w2, b2, w1sd, hn_ref, pt_ref):
    h = jnp.maximum(_dot(x_ref[...], w1[...]) + b1[...], 0.0)
    hn = _dot(h, w2[...]) + b2[...]
    hn_ref[...] = hn
    pt_ref[...] = _dot(hn, w1sd[...])


def _edge_step1_body(ea_ref, gs_ref, gd_ref, ew1, eb1, ew2, eb2,
                     w1e, b1, w2, b2, out_ref):
    eh = jnp.maximum(_dot(ea_ref[...], ew1[...]) + eb1[...], 0.0)
    he = _dot(eh, ew2[...]) + eb2[...]
    z = jnp.maximum(
        _dot(he, w1e[...]) + gs_ref[:, 0:_H] + gd_ref[:, _H:2 * _H]
        + b1[...], 0.0)
    en = _dot(z, w2[...]) + b2[...]
    out_ref[...] = jnp.concatenate([en, he + en], axis=1)


def _edge_step2_body(eh_ref, gs_ref, gd_ref, w1e, b1, w2, b2, out_ref):
    he = eh_ref[:, _H:2 * _H]
    z = jnp.maximum(
        _dot(he, w1e[...]) + gs_ref[:, 0:_H] + gd_ref[:, _H:2 * _H]
        + b1[...], 0.0)
    en = _dot(z, w2[...]) + b2[...]
    out_ref[...] = jnp.concatenate([en, jnp.zeros_like(en)], axis=1)


def _node_upd_body(hn_ref, parts_ref, w1h, w1a, b1, w2, b2, w1sd,
                   hn2_ref, pt_ref):
    agg = parts_ref[0, :, 0:_H] + parts_ref[1, :, 0:_H]
    h = jnp.maximum(
        _dot(hn_ref[...], w1h[...]) + _dot(agg, w1a[...]) + b1[...], 0.0)
    hn2 = hn_ref[...] + _dot(h, w2[...]) + b2[...]
    hn2_ref[...] = hn2
    pt_ref[...] = _dot(hn2, w1sd[...])


def _node_final_body(hn_ref, parts_ref, w1h, w1a, b1, w2, b2, dw1, db1,
                     dw2, db2, out_ref):
    agg = parts_ref[0, :, 0:_H] + parts_ref[1, :, 0:_H]
    h = jnp.maximum(
        _dot(hn_ref[...], w1h[...]) + _dot(agg, w1a[...]) + b1[...], 0.0)
    hn2 = hn_ref[...] + _dot(h, w2[...]) + b2[...]
    d = jnp.maximum(_dot(hn2, dw1[...]) + db1[...], 0.0)
    out_ref[...] = _dot(d, dw2[...]) + db2[...]


def kernel(x, edge_index, edge_attr, params):
    f32 = jnp.float32
    src = edge_index[0].astype(jnp.int32)
    dst = edge_index[1].astype(jnp.int32)
    src2 = src.reshape(_NW, _NCH, _C)
    dst2 = dst.reshape(_NW, _NCH, _C)
    zeros_tile = jnp.zeros((_NWB, 2 * _H), f32)

    pn, pe = params["enc_node"], params["enc_edge"]
    dec = params["dec_node"]

    def row(b):
        return b.reshape(1, -1)

    ew = [params["proc"][i]["edge"] for i in range(2)]
    nw = [params["proc"][i]["node"] for i in range(2)]
    w1sd = [jnp.concatenate([ew[i]["w1"][64:128], ew[i]["w1"][128:192]],
                            axis=1) for i in range(2)]

    grid_n = _N // _BN
    grid_e = _E // _BE
    bspec_n = pl.BlockSpec((_BN, _H), lambda i: (i, 0))
    bspec_n2 = pl.BlockSpec((_BN, 2 * _H), lambda i: (i, 0))
    bspec_e2 = pl.BlockSpec((_BE, 2 * _H), lambda i: (i, 0))
    w_spec = _full((_H, _H))
    wsd_spec = _full((_H, 2 * _H))
    b_spec = _full((1, _H))
    parts_spec = pl.BlockSpec((_NC, _BN, 2 * _H), lambda i: (0, i, 0))

    hn, pt1 = pl.pallas_call(
        _node_enc_body,
        grid=(grid_n,),
        in_specs=[pl.BlockSpec((_BN, 128), lambda i: (i, 0)),
                  _full((128, _H)), b_spec, w_spec, b_spec, wsd_spec],
        out_specs=[bspec_n, bspec_n2],
        out_shape=[jax.ShapeDtypeStruct((_N, _H), f32),
                   jax.ShapeDtypeStruct((_N, 2 * _H), f32)],
    )(x, pn["w1"], row(pn["b1"]), pn["w2"], row(pn["b2"]), w1sd[0])

    for step in range(2):
        gs, gd = _sc_gather(pt1, src2, dst2)
        if step == 0:
            enhen = pl.pallas_call(
                _edge_step1_body,
                grid=(grid_e,),
                in_specs=[pl.BlockSpec((_BE, 16), lambda i: (i, 0)),
                          bspec_e2, bspec_e2,
                          _full((16, _H)), b_spec, w_spec, b_spec,
                          w_spec, b_spec, w_spec, b_spec],
                out_specs=bspec_e2,
                out_shape=jax.ShapeDtypeStruct((_E, 2 * _H), f32),
            )(edge_attr, gs, gd, pe["w1"], row(pe["b1"]), pe["w2"],
              row(pe["b2"]), ew[0]["w1"][0:64], row(ew[0]["b1"]),
              ew[0]["w2"], row(ew[0]["b2"]))
        else:
            enhen = pl.pallas_call(
                _edge_step2_body,
                grid=(grid_e,),
                in_specs=[bspec_e2, bspec_e2, bspec_e2,
                          w_spec, b_spec, w_spec, b_spec],
                out_specs=bspec_e2,
                out_shape=jax.ShapeDtypeStruct((_E, 2 * _H), f32),
            )(prev_enhen, gs, gd, ew[1]["w1"][0:64], row(ew[1]["b1"]),
              ew[1]["w2"], row(ew[1]["b2"]))

        parts = _sc_scatter(enhen.reshape(_E // _C, _C, 2 * _H), dst2,
                            zeros_tile)
        prev_enhen = enhen

        if step == 0:
            hn, pt1 = pl.pallas_call(
                _node_upd_body,
                grid=(grid_n,),
                in_specs=[bspec_n, parts_spec, w_spec, w_spec, b_spec,
                          w_spec, b_spec, wsd_spec],
                out_specs=[bspec_n, bspec_n2],
                out_shape=[jax.ShapeDtypeStruct((_N, _H), f32),
                           jax.ShapeDtypeStruct((_N, 2 * _H), f32)],
            )(hn, parts, nw[0]["w1"][0:64], nw[0]["w1"][64:128],
              row(nw[0]["b1"]), nw[0]["w2"], row(nw[0]["b2"]), w1sd[1])
        else:
            out = pl.pallas_call(
                _node_final_body,
                grid=(grid_n,),
                in_specs=[bspec_n, parts_spec, w_spec, w_spec, b_spec,
                          w_spec, b_spec, w_spec, b_spec,
                          _full((_H, 4)), _full((1, 4))],
                out_specs=pl.BlockSpec((_BN, 4), lambda i: (i, 0)),
                out_shape=jax.ShapeDtypeStruct((_N, 4), f32),
            )(hn, parts, nw[1]["w1"][0:64], nw[1]["w1"][64:128],
              row(nw[1]["b1"]), nw[1]["w2"], row(nw[1]["b2"]),
              dec["w1"], row(dec["b1"]), dec["w2"], row(dec["b2"]))
    return out


# speedup vs baseline: 2.6377x; 1.0798x over previous
import functools

import jax
import jax.numpy as jnp
from jax import lax
from jax.experimental import pallas as pl
from jax.experimental.pallas import tpu as pltpu
from jax.experimental.pallas import tpu_sc as plsc

_N = 10000
_E = 320000
_H = 64

_NC = 2
_NS = 16
_NW = _NC * _NS
_PW = _E // _NW
_C = 80
_NCH = _PW // _C
_K = 5
_NRND = _NCH // _K
_HN = _N // 2
_NWB = 1000

_MESH = plsc.VectorSubcoreMesh(core_axis_name="c", subcore_axis_name="s")


@functools.partial(
    pl.kernel,
    out_type=[
        jax.ShapeDtypeStruct((_E, 2 * _H), jnp.float32),
        jax.ShapeDtypeStruct((_E, 2 * _H), jnp.float32),
    ],
    mesh=_MESH,
    scratch_types=[
        pltpu.VMEM((_NCH, _C), jnp.int32),
        pltpu.VMEM((2, _K, _C, 2 * _H), jnp.float32),
        pltpu.SemaphoreType.DMA,
        pltpu.SemaphoreType.DMA,
    ],
)
def _sc_gather(pt_hbm, src2_hbm, dst2_hbm, gs_out, gd_out,
               idx_ref, rows, gsem, ssem):
    wid = lax.axis_index("s") * _NC + lax.axis_index("c")
    base = wid * _PW

    def direction(idx2_hbm, out_ref):
        pltpu.sync_copy(idx2_hbm.at[wid], idx_ref)
        g0 = [pltpu.async_copy(pt_hbm.at[idx_ref.at[u]], rows.at[0, u],
                               gsem) for u in range(_K)]
        for h in g0:
            h.wait()

        def body(t, carry):
            b = lax.rem(t, 2)
            nb = 1 - b
            gs = [pltpu.async_copy(
                pt_hbm.at[idx_ref.at[t * _K + u]], rows.at[b, u], gsem)
                for u in range(_K)]
            ss = [pltpu.async_copy(
                rows.at[nb, u],
                out_ref.at[pl.ds(base + ((t - 1) * _K + u) * _C, _C)], ssem)
                for u in range(_K)]
            for h in ss:
                h.wait()
            for h in gs:
                h.wait()
            return carry

        lax.fori_loop(1, _NRND, body, 0)
        last = (_NRND - 1) % 2
        fs = [pltpu.async_copy(
            rows.at[last, u],
            out_ref.at[pl.ds(base + ((_NRND - 1) * _K + u) * _C, _C)], ssem)
            for u in range(_K)]
        for h in fs:
            h.wait()

    direction(src2_hbm, gs_out)
    direction(dst2_hbm, gd_out)


@functools.partial(
    pl.kernel,
    out_type=jax.ShapeDtypeStruct((_NC, _N, 2 * _H), jnp.float32),
    mesh=_MESH,
    scratch_types=[
        pltpu.VMEM((_NCH, _C), jnp.int32),
        pltpu.VMEM((_NCH, _C), jnp.int32),
        pltpu.VMEM((_K, _C, 2 * _H), jnp.float32),
        pltpu.VMEM_SHARED((_HN + 16, 2 * _H), jnp.float32),
        pltpu.SemaphoreType.DMA,
    ],
)
def _sc_scatter(en3_hbm, dst2_hbm, zeros_hbm, out_hbm,
                didx, didx2, rows, acc, lsem):
    c = lax.axis_index("c")
    s = lax.axis_index("s")
    wid = s * _NC + c
    pltpu.sync_copy(dst2_hbm.at[wid], didx)

    for p in range(2):
        lo = p * _HN

        @pl.when(s < _HN // _NWB)
        def _init():
            pltpu.sync_copy(zeros_hbm, acc.at[pl.ds(s * _NWB, _NWB)])

        @pl.when(s == _HN // _NWB)
        def _init_dump():
            pltpu.sync_copy(zeros_hbm.at[pl.ds(0, 16)],
                            acc.at[pl.ds(_HN, 16)])

        def remap_body(j, carry):
            for v in range(_C // 16):
                idx = didx[j, pl.ds(v * 16, 16)]
                ok = (idx >= lo) & (idx < lo + _HN)
                didx2[j, pl.ds(v * 16, 16)] = jnp.where(ok, idx - lo, _HN)
            return carry

        lax.fori_loop(0, _NCH, remap_body, 0)
        plsc.subcore_barrier()

        l0 = [pltpu.async_copy(en3_hbm.at[wid * _NCH + u], rows.at[u],
                               lsem) for u in range(_K)]
        for h in l0:
            h.wait()

        def round_body(t, carry):
            ls = []
            for u in range(_K):
                pltpu.sync_copy(rows.at[u],
                                acc.at[didx2.at[(t - 1) * _K + u]], add=True)
                ls.append(pltpu.async_copy(
                    en3_hbm.at[wid * _NCH + t * _K + u], rows.at[u], lsem))
            for h in ls:
                h.wait()
            return carry

        lax.fori_loop(1, _NRND, round_body, 0)
        for u in range(_K):
            pltpu.sync_copy(rows.at[u],
                            acc.at[didx2.at[(_NRND - 1) * _K + u]], add=True)
        plsc.subcore_barrier()

        @pl.when(s < _HN // _NWB)
        def _writeback():
            pltpu.sync_copy(acc.at[pl.ds(s * _NWB, _NWB)],
                            out_hbm.at[c, pl.ds(lo + s * _NWB, _NWB)])

        plsc.subcore_barrier()


_BN = 1000
_BE = 2000


def _dot(a, b):
    return jnp.dot(a, b, preferred_element_type=jnp.float32)


def _full(shape):
    return pl.BlockSpec(shape, lambda i: (0,) * len(shape))


def _node_enc_body(x_ref, w1, b1, w2, b2, w1sd, hn_ref, pt_ref):
    h = jnp.maximum(_dot(x_ref[...], w1[...]) + b1[...], 0.0)
    hn = _dot(h, w2[...]) + b2[...]
    hn_ref[...] = hn
    pt_ref[...] = _dot(hn, w1sd[...])


def _edge_step1_body(ea_ref, gs_ref, gd_ref, ew1, eb1, ew2, eb2,
                     w1e, b1, w2, b2, out_ref):
    eh = jnp.maximum(_dot(ea_ref[...], ew1[...]) + eb1[...], 0.0)
    he = _dot(eh, ew2[...]) + eb2[...]
    z = jnp.maximum(
        _dot(he, w1e[...]) + gs_ref[:, 0:_H] + gd_ref[:, _H:2 * _H]
        + b1[...], 0.0)
    en = _dot(z, w2[...]) + b2[...]
    out_ref[...] = jnp.concatenate([en, he + en], axis=1)


def _edge_step2_body(eh_ref, gs_ref, gd_ref, w1e, b1, w2, b2, out_ref):
    he = eh_ref[:, _H:2 * _H]
    z = jnp.maximum(
        _dot(he, w1e[...]) + gs_ref[:, 0:_H] + gd_ref[:, _H:2 * _H]
        + b1[...], 0.0)
    en = _dot(z, w2[...]) + b2[...]
    out_ref[...] = jnp.concatenate([en, jnp.zeros_like(en)], axis=1)


def _node_upd_body(hn_ref, parts_ref, w1h, w1a, b1, w2, b2, w1sd,
                   hn2_ref, pt_ref):
    agg = parts_ref[0, :, 0:_H] + parts_ref[1, :, 0:_H]
    h = jnp.maximum(
        _dot(hn_ref[...], w1h[...]) + _dot(agg, w1a[...]) + b1[...], 0.0)
    hn2 = hn_ref[...] + _dot(h, w2[...]) + b2[...]
    hn2_ref[...] = hn2
    pt_ref[...] = _dot(hn2, w1sd[...])


def _node_final_body(hn_ref, parts_ref, w1h, w1a, b1, w2, b2, dw1, db1,
                     dw2, db2, out_ref):
    agg = parts_ref[0, :, 0:_H] + parts_ref[1, :, 0:_H]
    h = jnp.maximum(
        _dot(hn_ref[...], w1h[...]) + _dot(agg, w1a[...]) + b1[...], 0.0)
    hn2 = hn_ref[...] + _dot(h, w2[...]) + b2[...]
    d = jnp.maximum(_dot(hn2, dw1[...]) + db1[...], 0.0)
    out_ref[...] = _dot(d, dw2[...]) + db2[...]


def kernel(x, edge_index, edge_attr, params):
    f32 = jnp.float32
    src = edge_index[0].astype(jnp.int32)
    dst = edge_index[1].astype(jnp.int32)
    src2 = src.reshape(_NW, _NCH, _C)
    dst2 = dst.reshape(_NW, _NCH, _C)
    zeros_tile = jnp.zeros((_NWB, 2 * _H), f32)

    pn, pe = params["enc_node"], params["enc_edge"]
    dec = params["dec_node"]

    def row(b):
        return b.reshape(1, -1)

    ew = [params["proc"][i]["edge"] for i in range(2)]
    nw = [params["proc"][i]["node"] for i in range(2)]
    w1sd = [jnp.concatenate([ew[i]["w1"][64:128], ew[i]["w1"][128:192]],
                            axis=1) for i in range(2)]

    grid_n = _N // _BN
    grid_e = _E // _BE
    bspec_n = pl.BlockSpec((_BN, _H), lambda i: (i, 0))
    bspec_n2 = pl.BlockSpec((_BN, 2 * _H), lambda i: (i, 0))
    bspec_e2 = pl.BlockSpec((_BE, 2 * _H), lambda i: (i, 0))
    w_spec = _full((_H, _H))
    wsd_spec = _full((_H, 2 * _H))
    b_spec = _full((1, _H))
    parts_spec = pl.BlockSpec((_NC, _BN, 2 * _H), lambda i: (0, i, 0))

    hn, pt1 = pl.pallas_call(
        _node_enc_body,
        grid=(grid_n,),
        in_specs=[pl.BlockSpec((_BN, 128), lambda i: (i, 0)),
                  _full((128, _H)), b_spec, w_spec, b_spec, wsd_spec],
        out_specs=[bspec_n, bspec_n2],
        out_shape=[jax.ShapeDtypeStruct((_N, _H), f32),
                   jax.ShapeDtypeStruct((_N, 2 * _H), f32)],
    )(x, pn["w1"], row(pn["b1"]), pn["w2"], row(pn["b2"]), w1sd[0])

    for step in range(2):
        gs, gd = _sc_gather(pt1, src2, dst2)
        if step == 0:
            enhen = pl.pallas_call(
                _edge_step1_body,
                grid=(grid_e,),
                in_specs=[pl.BlockSpec((_BE, 16), lambda i: (i, 0)),
                          bspec_e2, bspec_e2,
                          _full((16, _H)), b_spec, w_spec, b_spec,
                          w_spec, b_spec, w_spec, b_spec],
                out_specs=bspec_e2,
                out_shape=jax.ShapeDtypeStruct((_E, 2 * _H), f32),
            )(edge_attr, gs, gd, pe["w1"], row(pe["b1"]), pe["w2"],
              row(pe["b2"]), ew[0]["w1"][0:64], row(ew[0]["b1"]),
              ew[0]["w2"], row(ew[0]["b2"]))
        else:
            enhen = pl.pallas_call(
                _edge_step2_body,
                grid=(grid_e,),
                in_specs=[bspec_e2, bspec_e2, bspec_e2,
                          w_spec, b_spec, w_spec, b_spec],
                out_specs=bspec_e2,
                out_shape=jax.ShapeDtypeStruct((_E, 2 * _H), f32),
            )(prev_enhen, gs, gd, ew[1]["w1"][0:64], row(ew[1]["b1"]),
              ew[1]["w2"], row(ew[1]["b2"]))

        parts = _sc_scatter(enhen.reshape(_E // _C, _C, 2 * _H), dst2,
                            zeros_tile)
        prev_enhen = enhen

        if step == 0:
            hn, pt1 = pl.pallas_call(
                _node_upd_body,
                grid=(grid_n,),
                in_specs=[bspec_n, parts_spec, w_spec, w_spec, b_spec,
                          w_spec, b_spec, wsd_spec],
                out_specs=[bspec_n, bspec_n2],
                out_shape=[jax.ShapeDtypeStruct((_N, _H), f32),
                           jax.ShapeDtypeStruct((_N, 2 * _H), f32)],
            )(hn, parts, nw[0]["w1"][0:64], nw[0]["w1"][64:128],
              row(nw[0]["b1"]), nw[0]["w2"], row(nw[0]["b2"]), w1sd[1])
        else:
            out = pl.pallas_call(
                _node_final_body,
                grid=(grid_n,),
                in_specs=[bspec_n, parts_spec, w_spec, w_spec, b_spec,
                          w_spec, b_spec, w_spec, b_spec,
                          _full((_H, 4)), _full((1, 4))],
                out_specs=pl.BlockSpec((_BN, 4), lambda i: (i, 0)),
                out_shape=jax.ShapeDtypeStruct((_N, 4), f32),
            )(hn, parts, nw[1]["w1"][0:64], nw[1]["w1"][64:128],
              row(nw[1]["b1"]), nw[1]["w2"], row(nw[1]["b2"]),
              dec["w1"], row(dec["b1"]), dec["w2"], row(dec["b2"]))
    return out


# speedup vs baseline: 2.8309x; 1.0733x over previous
import functools

import jax
import jax.numpy as jnp
from jax import lax
from jax.experimental import pallas as pl
from jax.experimental.pallas import tpu as pltpu
from jax.experimental.pallas import tpu_sc as plsc

_N = 10000
_E = 320000
_H = 64

_NC = 2
_NS = 16
_NW = _NC * _NS
_PW = _E // _NW
_C = 80
_NCH = _PW // _C
_K = 5
_NRND = _NCH // _K
_HN = _N // 2
_NWB = 1000

_MESH = plsc.VectorSubcoreMesh(core_axis_name="c", subcore_axis_name="s")


@functools.partial(
    pl.kernel,
    out_type=[
        jax.ShapeDtypeStruct((_E, 2 * _H), jnp.float32),
        jax.ShapeDtypeStruct((_E, 2 * _H), jnp.float32),
    ],
    mesh=_MESH,
    scratch_types=[
        pltpu.VMEM((_NCH, _C), jnp.int32),
        pltpu.VMEM((2, _K, _C, 2 * _H), jnp.float32),
        pltpu.SemaphoreType.DMA,
        pltpu.SemaphoreType.DMA,
    ],
)
def _sc_gather(pt_hbm, src2_hbm, dst2_hbm, gs_out, gd_out,
               idx_ref, rows, gsem, ssem):
    wid = lax.axis_index("s") * _NC + lax.axis_index("c")
    base = wid * _PW

    def direction(idx2_hbm, out_ref):
        pltpu.sync_copy(idx2_hbm.at[wid], idx_ref)
        g0 = [pltpu.async_copy(pt_hbm.at[idx_ref.at[u]], rows.at[0, u],
                               gsem) for u in range(_K)]
        for h in g0:
            h.wait()

        def body(t, carry):
            b = lax.rem(t, 2)
            nb = 1 - b
            gs = [pltpu.async_copy(
                pt_hbm.at[idx_ref.at[t * _K + u]], rows.at[b, u], gsem)
                for u in range(_K)]
            ss = [pltpu.async_copy(
                rows.at[nb, u],
                out_ref.at[pl.ds(base + ((t - 1) * _K + u) * _C, _C)], ssem)
                for u in range(_K)]
            for h in ss:
                h.wait()
            for h in gs:
                h.wait()
            return carry

        lax.fori_loop(1, _NRND, body, 0)
        last = (_NRND - 1) % 2
        fs = [pltpu.async_copy(
            rows.at[last, u],
            out_ref.at[pl.ds(base + ((_NRND - 1) * _K + u) * _C, _C)], ssem)
            for u in range(_K)]
        for h in fs:
            h.wait()

    direction(src2_hbm, gs_out)
    direction(dst2_hbm, gd_out)


@functools.partial(
    pl.kernel,
    out_type=jax.ShapeDtypeStruct((_NC, _N, 2 * _H), jnp.float32),
    mesh=_MESH,
    scratch_types=[
        pltpu.VMEM((_NCH, _C), jnp.int32),
        pltpu.VMEM((_NCH, _C), jnp.int32),
        pltpu.VMEM((_K, _C, 2 * _H), jnp.float32),
        pltpu.VMEM_SHARED((_HN + 16, 2 * _H), jnp.float32),
        pltpu.SemaphoreType.DMA,
        pltpu.SemaphoreType.DMA,
    ],
)
def _sc_scatter(en3_hbm, dst2_hbm, zeros_hbm, out_hbm,
                didx, didx2, rows, acc, lsem, asem):
    c = lax.axis_index("c")
    s = lax.axis_index("s")
    wid = s * _NC + c
    pltpu.sync_copy(dst2_hbm.at[wid], didx)

    for p in range(2):
        lo = p * _HN

        @pl.when(s < _HN // _NWB)
        def _init():
            pltpu.sync_copy(zeros_hbm, acc.at[pl.ds(s * _NWB, _NWB)])

        @pl.when(s == _HN // _NWB)
        def _init_dump():
            pltpu.sync_copy(zeros_hbm.at[pl.ds(0, 16)],
                            acc.at[pl.ds(_HN, 16)])

        def remap_body(j, carry):
            for v in range(_C // 16):
                idx = didx[j, pl.ds(v * 16, 16)]
                ok = (idx >= lo) & (idx < lo + _HN)
                didx2[j, pl.ds(v * 16, 16)] = jnp.where(ok, idx - lo, _HN)
            return carry

        lax.fori_loop(0, _NCH, remap_body, 0)
        plsc.subcore_barrier()

        l0 = [pltpu.async_copy(en3_hbm.at[wid * _NCH + u], rows.at[u],
                               lsem) for u in range(_K)]
        for h in l0:
            h.wait()

        def round_body(t, carry):
            ads = [pltpu.async_copy(
                rows.at[u], acc.at[didx2.at[(t - 1) * _K + u]], asem,
                add=True) for u in range(_K)]
            ls = []
            for u in range(_K):
                ads[u].wait()
                ls.append(pltpu.async_copy(
                    en3_hbm.at[wid * _NCH + t * _K + u], rows.at[u], lsem))
            for h in ls:
                h.wait()
            return carry

        lax.fori_loop(1, _NRND, round_body, 0)
        fs = [pltpu.async_copy(
            rows.at[u], acc.at[didx2.at[(_NRND - 1) * _K + u]], asem,
            add=True) for u in range(_K)]
        for h in fs:
            h.wait()
        plsc.subcore_barrier()

        @pl.when(s < _HN // _NWB)
        def _writeback():
            pltpu.sync_copy(acc.at[pl.ds(s * _NWB, _NWB)],
                            out_hbm.at[c, pl.ds(lo + s * _NWB, _NWB)])

        plsc.subcore_barrier()


_BN = 2000
_BE = 4000


def _dot(a, b):
    return jnp.dot(a, b, preferred_element_type=jnp.float32)


def _full(shape):
    return pl.BlockSpec(shape, lambda i: (0,) * len(shape))


def _node_enc_body(x_ref, w1, b1, w2, b2, w1sd, hn_ref, pt_ref):
    h = jnp.maximum(_dot(x_ref[...], w1[...]) + b1[...], 0.0)
    hn = _dot(h, w2[...]) + b2[...]
    hn_ref[...] = hn
    pt_ref[...] = _dot(hn, w1sd[...])


def _edge_step1_body(ea_ref, gs_ref, gd_ref, ew1, eb1, ew2, eb2,
                     w1e, b1, w2, b2, out_ref):
    eh = jnp.maximum(_dot(ea_ref[...], ew1[...]) + eb1[...], 0.0)
    he = _dot(eh, ew2[...]) + eb2[...]
    z = jnp.maximum(
        _dot(he, w1e[...]) + gs_ref[:, 0:_H] + gd_ref[:, _H:2 * _H]
        + b1[...], 0.0)
    en = _dot(z, w2[...]) + b2[...]
    out_ref[...] = jnp.concatenate([en, he + en], axis=1)


def _edge_step2_body(eh_ref, gs_ref, gd_ref, w1e, b1, w2, b2, out_ref):
    he = eh_ref[:, _H:2 * _H]
    z = jnp.maximum(
        _dot(he, w1e[...]) + gs_ref[:, 0:_H] + gd_ref[:, _H:2 * _H]
        + b1[...], 0.0)
    en = _dot(z, w2[...]) + b2[...]
    out_ref[...] = jnp.concatenate([en, jnp.zeros_like(en)], axis=1)


def _node_upd_body(hn_ref, parts_ref, w1h, w1a, b1, w2, b2, w1sd,
                   hn2_ref, pt_ref):
    agg = parts_ref[0, :, 0:_H] + parts_ref[1, :, 0:_H]
    h = jnp.maximum(
        _dot(hn_ref[...], w1h[...]) + _dot(agg, w1a[...]) + b1[...], 0.0)
    hn2 = hn_ref[...] + _dot(h, w2[...]) + b2[...]
    hn2_ref[...] = hn2
    pt_ref[...] = _dot(hn2, w1sd[...])


def _node_final_body(hn_ref, parts_ref, w1h, w1a, b1, w2, b2, dw1, db1,
                     dw2, db2, out_ref):
    agg = parts_ref[0, :, 0:_H] + parts_ref[1, :, 0:_H]
    h = jnp.maximum(
        _dot(hn_ref[...], w1h[...]) + _dot(agg, w1a[...]) + b1[...], 0.0)
    hn2 = hn_ref[...] + _dot(h, w2[...]) + b2[...]
    d = jnp.maximum(_dot(hn2, dw1[...]) + db1[...], 0.0)
    out_ref[...] = _dot(d, dw2[...]) + db2[...]


def kernel(x, edge_index, edge_attr, params):
    f32 = jnp.float32
    src = edge_index[0].astype(jnp.int32)
    dst = edge_index[1].astype(jnp.int32)
    src2 = src.reshape(_NW, _NCH, _C)
    dst2 = dst.reshape(_NW, _NCH, _C)
    zeros_tile = jnp.zeros((_NWB, 2 * _H), f32)

    pn, pe = params["enc_node"], params["enc_edge"]
    dec = params["dec_node"]

    def row(b):
        return b.reshape(1, -1)

    ew = [params["proc"][i]["edge"] for i in range(2)]
    nw = [params["proc"][i]["node"] for i in range(2)]
    w1sd = [jnp.concatenate([ew[i]["w1"][64:128], ew[i]["w1"][128:192]],
                            axis=1) for i in range(2)]

    grid_n = _N // _BN
    grid_e = _E // _BE
    bspec_n = pl.BlockSpec((_BN, _H), lambda i: (i, 0))
    bspec_n2 = pl.BlockSpec((_BN, 2 * _H), lambda i: (i, 0))
    bspec_e2 = pl.BlockSpec((_BE, 2 * _H), lambda i: (i, 0))
    w_spec = _full((_H, _H))
    wsd_spec = _full((_H, 2 * _H))
    b_spec = _full((1, _H))
    parts_spec = pl.BlockSpec((_NC, _BN, 2 * _H), lambda i: (0, i, 0))

    hn, pt1 = pl.pallas_call(
        _node_enc_body,
        grid=(grid_n,),
        in_specs=[pl.BlockSpec((_BN, 128), lambda i: (i, 0)),
                  _full((128, _H)), b_spec, w_spec, b_spec, wsd_spec],
        out_specs=[bspec_n, bspec_n2],
        out_shape=[jax.ShapeDtypeStruct((_N, _H), f32),
                   jax.ShapeDtypeStruct((_N, 2 * _H), f32)],
    )(x, pn["w1"], row(pn["b1"]), pn["w2"], row(pn["b2"]), w1sd[0])

    for step in range(2):
        gs, gd = _sc_gather(pt1, src2, dst2)
        if step == 0:
            enhen = pl.pallas_call(
                _edge_step1_body,
                grid=(grid_e,),
                in_specs=[pl.BlockSpec((_BE, 16), lambda i: (i, 0)),
                          bspec_e2, bspec_e2,
                          _full((16, _H)), b_spec, w_spec, b_spec,
                          w_spec, b_spec, w_spec, b_spec],
                out_specs=bspec_e2,
                out_shape=jax.ShapeDtypeStruct((_E, 2 * _H), f32),
            )(edge_attr, gs, gd, pe["w1"], row(pe["b1"]), pe["w2"],
              row(pe["b2"]), ew[0]["w1"][0:64], row(ew[0]["b1"]),
              ew[0]["w2"], row(ew[0]["b2"]))
        else:
            enhen = pl.pallas_call(
                _edge_step2_body,
                grid=(grid_e,),
                in_specs=[bspec_e2, bspec_e2, bspec_e2,
                          w_spec, b_spec, w_spec, b_spec],
                out_specs=bspec_e2,
                out_shape=jax.ShapeDtypeStruct((_E, 2 * _H), f32),
            )(prev_enhen, gs, gd, ew[1]["w1"][0:64], row(ew[1]["b1"]),
              ew[1]["w2"], row(ew[1]["b2"]))

        parts = _sc_scatter(enhen.reshape(_E // _C, _C, 2 * _H), dst2,
                            zeros_tile)
        prev_enhen = enhen

        if step == 0:
            hn, pt1 = pl.pallas_call(
                _node_upd_body,
                grid=(grid_n,),
                in_specs=[bspec_n, parts_spec, w_spec, w_spec, b_spec,
                          w_spec, b_spec, wsd_spec],
                out_specs=[bspec_n, bspec_n2],
                out_shape=[jax.ShapeDtypeStruct((_N, _H), f32),
                           jax.ShapeDtypeStruct((_N, 2 * _H), f32)],
            )(hn, parts, nw[0]["w1"][0:64], nw[0]["w1"][64:128],
              row(nw[0]["b1"]), nw[0]["w2"], row(nw[0]["b2"]), w1sd[1])
        else:
            out = pl.pallas_call(
                _node_final_body,
                grid=(grid_n,),
                in_specs=[bspec_n, parts_spec, w_spec, w_spec, b_spec,
                          w_spec, b_spec, w_spec, b_spec,
                          _full((_H, 4)), _full((1, 4))],
                out_specs=pl.BlockSpec((_BN, 4), lambda i: (i, 0)),
                out_shape=jax.ShapeDtypeStruct((_N, 4), f32),
            )(hn, parts, nw[1]["w1"][0:64], nw[1]["w1"][64:128],
              row(nw[1]["b1"]), nw[1]["w2"], row(nw[1]["b2"]),
              dec["w1"], row(dec["b1"]), dec["w2"], row(dec["b2"]))
    return out


# speedup vs baseline: 3.4609x; 1.2225x over previous
import functools

import jax
import jax.numpy as jnp
from jax import lax
from jax.experimental import pallas as pl
from jax.experimental.pallas import tpu as pltpu
from jax.experimental.pallas import tpu_sc as plsc

_N = 10000
_E = 320000
_H = 64

_NC = 2
_NS = 16
_NW = _NC * _NS
_PW = _E // _NW
_C = 80
_NCH = _PW // _C
_K = 5
_NRND = _NCH // _K
_HN = _N // 2
_NWB = 1000

_MESH = plsc.VectorSubcoreMesh(core_axis_name="c", subcore_axis_name="s")


@functools.partial(
    pl.kernel,
    out_type=[
        jax.ShapeDtypeStruct((_E, 2 * _H), jnp.float32),
        jax.ShapeDtypeStruct((_E, 2 * _H), jnp.float32),
    ],
    mesh=_MESH,
    scratch_types=[
        pltpu.VMEM((_NCH, _C), jnp.int32),
        pltpu.VMEM((2, _K, _C, 2 * _H), jnp.float32),
        pltpu.SemaphoreType.DMA,
        pltpu.SemaphoreType.DMA,
    ],
)
def _sc_gather(pt_hbm, src2_hbm, dst2_hbm, gs_out, gd_out,
               idx_ref, rows, gsem, ssem):
    wid = lax.axis_index("s") * _NC + lax.axis_index("c")
    base = wid * _PW

    def direction(idx2_hbm, out_ref):
        pltpu.sync_copy(idx2_hbm.at[wid], idx_ref)
        g0 = [pltpu.async_copy(pt_hbm.at[idx_ref.at[u]], rows.at[0, u],
                               gsem) for u in range(_K)]
        for h in g0:
            h.wait()

        def body(t, carry):
            b = lax.rem(t, 2)
            nb = 1 - b
            gs = [pltpu.async_copy(
                pt_hbm.at[idx_ref.at[t * _K + u]], rows.at[b, u], gsem)
                for u in range(_K)]
            ss = [pltpu.async_copy(
                rows.at[nb, u],
                out_ref.at[pl.ds(base + ((t - 1) * _K + u) * _C, _C)], ssem)
                for u in range(_K)]
            for h in ss:
                h.wait()
            for h in gs:
                h.wait()
            return carry

        lax.fori_loop(1, _NRND, body, 0)
        last = (_NRND - 1) % 2
        fs = [pltpu.async_copy(
            rows.at[last, u],
            out_ref.at[pl.ds(base + ((_NRND - 1) * _K + u) * _C, _C)], ssem)
            for u in range(_K)]
        for h in fs:
            h.wait()

    direction(src2_hbm, gs_out)
    direction(dst2_hbm, gd_out)


_KS = 3
_NRS = (_NCH - 1) // _KS
_NSIO = _N // _NWB


@functools.partial(
    pl.kernel,
    out_type=jax.ShapeDtypeStruct((_NC, _N, 2 * _H), jnp.float32),
    mesh=_MESH,
    scratch_types=[
        pltpu.VMEM((_NCH, _C), jnp.int32),
        pltpu.VMEM((_KS, _C, 2 * _H), jnp.float32),
        pltpu.VMEM_SHARED((_N, 2 * _H), jnp.float32),
        pltpu.SemaphoreType.DMA,
        pltpu.SemaphoreType.DMA,
    ],
)
def _sc_scatter(en3_hbm, dst2_hbm, zeros_hbm, out_hbm,
                didx, rows, acc, lsem, asem):
    c = lax.axis_index("c")
    s = lax.axis_index("s")
    wid = s * _NC + c
    pltpu.sync_copy(dst2_hbm.at[wid], didx)

    @pl.when(s < _NSIO)
    def _init():
        pltpu.sync_copy(zeros_hbm, acc.at[pl.ds(s * _NWB, _NWB)])

    plsc.subcore_barrier()

    l0 = [pltpu.async_copy(en3_hbm.at[wid * _NCH + u], rows.at[u], lsem)
          for u in range(_KS)]
    for h in l0:
        h.wait()

    def round_body(t, carry):
        ads = [pltpu.async_copy(
            rows.at[u], acc.at[didx.at[(t - 1) * _KS + u]], asem,
            add=True) for u in range(_KS)]
        ls = []
        for u in range(_KS):
            ads[u].wait()
            ls.append(pltpu.async_copy(
                en3_hbm.at[wid * _NCH + t * _KS + u], rows.at[u], lsem))
        for h in ls:
            h.wait()
        return carry

    lax.fori_loop(1, _NRS, round_body, 0)
    fs = [pltpu.async_copy(
        rows.at[u], acc.at[didx.at[(_NRS - 1) * _KS + u]], asem,
        add=True) for u in range(_KS)]
    for h in fs:
        h.wait()
    for j in range(_NRS * _KS, _NCH):
        pltpu.sync_copy(en3_hbm.at[wid * _NCH + j], rows.at[0])
        pltpu.sync_copy(rows.at[0], acc.at[didx.at[j]], add=True)
    plsc.subcore_barrier()

    @pl.when(s < _NSIO)
    def _writeback():
        pltpu.sync_copy(acc.at[pl.ds(s * _NWB, _NWB)],
                        out_hbm.at[c, pl.ds(s * _NWB, _NWB)])


_BN = 2000
_BE = 4000


def _dot(a, b):
    return jnp.dot(a, b, preferred_element_type=jnp.float32)


def _full(shape):
    return pl.BlockSpec(shape, lambda i: (0,) * len(shape))


def _node_enc_body(x_ref, w1, b1, w2, b2, w1sd, hn_ref, pt_ref):
    h = jnp.maximum(_dot(x_ref[...], w1[...]) + b1[...], 0.0)
    hn = _dot(h, w2[...]) + b2[...]
    hn_ref[...] = hn
    pt_ref[...] = _dot(hn, w1sd[...])


def _edge_step1_body(ea_ref, gs_ref, gd_ref, ew1, eb1, ew2, eb2,
                     w1e, b1, w2, b2, out_ref):
    eh = jnp.maximum(_dot(ea_ref[...], ew1[...]) + eb1[...], 0.0)
    he = _dot(eh, ew2[...]) + eb2[...]
    z = jnp.maximum(
        _dot(he, w1e[...]) + gs_ref[:, 0:_H] + gd_ref[:, _H:2 * _H]
        + b1[...], 0.0)
    en = _dot(z, w2[...]) + b2[...]
    out_ref[...] = jnp.concatenate([en, he + en], axis=1)


def _edge_step2_body(eh_ref, gs_ref, gd_ref, w1e, b1, w2, b2, out_ref):
    he = eh_ref[:, _H:2 * _H]
    z = jnp.maximum(
        _dot(he, w1e[...]) + gs_ref[:, 0:_H] + gd_ref[:, _H:2 * _H]
        + b1[...], 0.0)
    en = _dot(z, w2[...]) + b2[...]
    out_ref[...] = jnp.concatenate([en, jnp.zeros_like(en)], axis=1)


def _node_upd_body(hn_ref, parts_ref, w1h, w1a, b1, w2, b2, w1sd,
                   hn2_ref, pt_ref):
    agg = parts_ref[0, :, 0:_H] + parts_ref[1, :, 0:_H]
    h = jnp.maximum(
        _dot(hn_ref[...], w1h[...]) + _dot(agg, w1a[...]) + b1[...], 0.0)
    hn2 = hn_ref[...] + _dot(h, w2[...]) + b2[...]
    hn2_ref[...] = hn2
    pt_ref[...] = _dot(hn2, w1sd[...])


def _node_final_body(hn_ref, parts_ref, w1h, w1a, b1, w2, b2, dw1, db1,
                     dw2, db2, out_ref):
    agg = parts_ref[0, :, 0:_H] + parts_ref[1, :, 0:_H]
    h = jnp.maximum(
        _dot(hn_ref[...], w1h[...]) + _dot(agg, w1a[...]) + b1[...], 0.0)
    hn2 = hn_ref[...] + _dot(h, w2[...]) + b2[...]
    d = jnp.maximum(_dot(hn2, dw1[...]) + db1[...], 0.0)
    out_ref[...] = _dot(d, dw2[...]) + db2[...]


def kernel(x, edge_index, edge_attr, params):
    f32 = jnp.float32
    src = edge_index[0].astype(jnp.int32)
    dst = edge_index[1].astype(jnp.int32)
    src2 = src.reshape(_NW, _NCH, _C)
    dst2 = dst.reshape(_NW, _NCH, _C)
    zeros_tile = jnp.zeros((_NWB, 2 * _H), f32)

    pn, pe = params["enc_node"], params["enc_edge"]
    dec = params["dec_node"]

    def row(b):
        return b.reshape(1, -1)

    ew = [params["proc"][i]["edge"] for i in range(2)]
    nw = [params["proc"][i]["node"] for i in range(2)]
    w1sd = [jnp.concatenate([ew[i]["w1"][64:128], ew[i]["w1"][128:192]],
                            axis=1) for i in range(2)]

    grid_n = _N // _BN
    grid_e = _E // _BE
    bspec_n = pl.BlockSpec((_BN, _H), lambda i: (i, 0))
    bspec_n2 = pl.BlockSpec((_BN, 2 * _H), lambda i: (i, 0))
    bspec_e2 = pl.BlockSpec((_BE, 2 * _H), lambda i: (i, 0))
    w_spec = _full((_H, _H))
    wsd_spec = _full((_H, 2 * _H))
    b_spec = _full((1, _H))
    parts_spec = pl.BlockSpec((_NC, _BN, 2 * _H), lambda i: (0, i, 0))

    hn, pt1 = pl.pallas_call(
        _node_enc_body,
        grid=(grid_n,),
        in_specs=[pl.BlockSpec((_BN, 128), lambda i: (i, 0)),
                  _full((128, _H)), b_spec, w_spec, b_spec, wsd_spec],
        out_specs=[bspec_n, bspec_n2],
        out_shape=[jax.ShapeDtypeStruct((_N, _H), f32),
                   jax.ShapeDtypeStruct((_N, 2 * _H), f32)],
    )(x, pn["w1"], row(pn["b1"]), pn["w2"], row(pn["b2"]), w1sd[0])

    for step in range(2):
        gs, gd = _sc_gather(pt1, src2, dst2)
        if step == 0:
            enhen = pl.pallas_call(
                _edge_step1_body,
                grid=(grid_e,),
                in_specs=[pl.BlockSpec((_BE, 16), lambda i: (i, 0)),
                          bspec_e2, bspec_e2,
                          _full((16, _H)), b_spec, w_spec, b_spec,
                          w_spec, b_spec, w_spec, b_spec],
                out_specs=bspec_e2,
                out_shape=jax.ShapeDtypeStruct((_E, 2 * _H), f32),
            )(edge_attr, gs, gd, pe["w1"], row(pe["b1"]), pe["w2"],
              row(pe["b2"]), ew[0]["w1"][0:64], row(ew[0]["b1"]),
              ew[0]["w2"], row(ew[0]["b2"]))
        else:
            enhen = pl.pallas_call(
                _edge_step2_body,
                grid=(grid_e,),
                in_specs=[bspec_e2, bspec_e2, bspec_e2,
                          w_spec, b_spec, w_spec, b_spec],
                out_specs=bspec_e2,
                out_shape=jax.ShapeDtypeStruct((_E, 2 * _H), f32),
            )(prev_enhen, gs, gd, ew[1]["w1"][0:64], row(ew[1]["b1"]),
              ew[1]["w2"], row(ew[1]["b2"]))

        parts = _sc_scatter(enhen.reshape(_E // _C, _C, 2 * _H), dst2,
                            zeros_tile)
        prev_enhen = enhen

        if step == 0:
            hn, pt1 = pl.pallas_call(
                _node_upd_body,
                grid=(grid_n,),
                in_specs=[bspec_n, parts_spec, w_spec, w_spec, b_spec,
                          w_spec, b_spec, wsd_spec],
                out_specs=[bspec_n, bspec_n2],
                out_shape=[jax.ShapeDtypeStruct((_N, _H), f32),
                           jax.ShapeDtypeStruct((_N, 2 * _H), f32)],
            )(hn, parts, nw[0]["w1"][0:64], nw[0]["w1"][64:128],
              row(nw[0]["b1"]), nw[0]["w2"], row(nw[0]["b2"]), w1sd[1])
        else:
            out = pl.pallas_call(
                _node_final_body,
                grid=(grid_n,),
                in_specs=[bspec_n, parts_spec, w_spec, w_spec, b_spec,
                          w_spec, b_spec, w_spec, b_spec,
                          _full((_H, 4)), _full((1, 4))],
                out_specs=pl.BlockSpec((_BN, 4), lambda i: (i, 0)),
                out_shape=jax.ShapeDtypeStruct((_N, 4), f32),
            )(hn, parts, nw[1]["w1"][0:64], nw[1]["w1"][64:128],
              row(nw[1]["b1"]), nw[1]["w2"], row(nw[1]["b2"]),
              dec["w1"], row(dec["b1"]), dec["w2"], row(dec["b2"]))
    return out


# speedup vs baseline: 3.4674x; 1.0019x over previous
import functools

import jax
import jax.numpy as jnp
from jax import lax
from jax.experimental import pallas as pl
from jax.experimental.pallas import tpu as pltpu
from jax.experimental.pallas import tpu_sc as plsc

_N = 10000
_E = 320000
_H = 64

_NC = 2
_NS = 16
_NW = _NC * _NS
_PW = _E // _NW
_C = 80
_NCH = _PW // _C
_K = 5
_NRND = _NCH // _K
_HN = _N // 2
_NWB = 1000

_MESH = plsc.VectorSubcoreMesh(core_axis_name="c", subcore_axis_name="s")


@functools.partial(
    pl.kernel,
    out_type=[
        jax.ShapeDtypeStruct((_E, 2 * _H), jnp.float32),
        jax.ShapeDtypeStruct((_E, 2 * _H), jnp.float32),
    ],
    mesh=_MESH,
    scratch_types=[
        pltpu.VMEM((_NCH, _C), jnp.int32),
        pltpu.VMEM((2, _K, _C, 2 * _H), jnp.float32),
        pltpu.SemaphoreType.DMA,
        pltpu.SemaphoreType.DMA,
    ],
)
def _sc_gather(pt_hbm, src2_hbm, dst2_hbm, gs_out, gd_out,
               idx_ref, rows, gsem, ssem):
    wid = lax.axis_index("s") * _NC + lax.axis_index("c")
    base = wid * _PW

    def direction(idx2_hbm, out_ref):
        pltpu.sync_copy(idx2_hbm.at[wid], idx_ref)
        g0 = [pltpu.async_copy(pt_hbm.at[idx_ref.at[u]], rows.at[0, u],
                               gsem) for u in range(_K)]
        for h in g0:
            h.wait()

        def body(t, carry):
            b = lax.rem(t, 2)
            nb = 1 - b
            gs = [pltpu.async_copy(
                pt_hbm.at[idx_ref.at[t * _K + u]], rows.at[b, u], gsem)
                for u in range(_K)]
            ss = [pltpu.async_copy(
                rows.at[nb, u],
                out_ref.at[pl.ds(base + ((t - 1) * _K + u) * _C, _C)], ssem)
                for u in range(_K)]
            for h in ss:
                h.wait()
            for h in gs:
                h.wait()
            return carry

        lax.fori_loop(1, _NRND, body, 0)
        last = (_NRND - 1) % 2
        fs = [pltpu.async_copy(
            rows.at[last, u],
            out_ref.at[pl.ds(base + ((_NRND - 1) * _K + u) * _C, _C)], ssem)
            for u in range(_K)]
        for h in fs:
            h.wait()

    direction(src2_hbm, gs_out)
    direction(dst2_hbm, gd_out)


_KS = 3
_NRS = (_NCH - 1) // _KS
_NSIO = _N // _NWB


@functools.partial(
    pl.kernel,
    out_type=jax.ShapeDtypeStruct((_NC, _N, 2 * _H), jnp.float32),
    mesh=_MESH,
    scratch_types=[
        pltpu.VMEM((_NCH, _C), jnp.int32),
        pltpu.VMEM((_KS, _C, 2 * _H), jnp.float32),
        pltpu.VMEM_SHARED((_N, 2 * _H), jnp.float32),
        pltpu.SemaphoreType.DMA,
        pltpu.SemaphoreType.DMA,
    ],
)
def _sc_scatter(en3_hbm, dst2_hbm, zeros_hbm, out_hbm,
                didx, rows, acc, lsem, asem):
    c = lax.axis_index("c")
    s = lax.axis_index("s")
    wid = s * _NC + c
    pltpu.sync_copy(dst2_hbm.at[wid], didx)

    @pl.when(s < _NSIO)
    def _init():
        pltpu.sync_copy(zeros_hbm, acc.at[pl.ds(s * _NWB, _NWB)])

    plsc.subcore_barrier()

    l0 = [pltpu.async_copy(en3_hbm.at[wid * _NCH + u], rows.at[u], lsem)
          for u in range(_KS)]
    for h in l0:
        h.wait()

    def round_body(t, carry):
        ads = [pltpu.async_copy(
            rows.at[u], acc.at[didx.at[(t - 1) * _KS + u]], asem,
            add=True) for u in range(_KS)]
        ls = []
        for u in range(_KS):
            ads[u].wait()
            ls.append(pltpu.async_copy(
                en3_hbm.at[wid * _NCH + t * _KS + u], rows.at[u], lsem))
        for h in ls:
            h.wait()
        return carry

    lax.fori_loop(1, _NRS, round_body, 0)
    fs = [pltpu.async_copy(
        rows.at[u], acc.at[didx.at[(_NRS - 1) * _KS + u]], asem,
        add=True) for u in range(_KS)]
    for h in fs:
        h.wait()
    for j in range(_NRS * _KS, _NCH):
        pltpu.sync_copy(en3_hbm.at[wid * _NCH + j], rows.at[0])
        pltpu.sync_copy(rows.at[0], acc.at[didx.at[j]], add=True)
    plsc.subcore_barrier()

    @pl.when(s < _NSIO)
    def _writeback():
        pltpu.sync_copy(acc.at[pl.ds(s * _NWB, _NWB)],
                        out_hbm.at[c, pl.ds(s * _NWB, _NWB)])


_BN = 2000
_BE = 4000


def _dot(a, b):
    return jnp.dot(a, b, preferred_element_type=jnp.float32)


def _full(shape):
    return pl.BlockSpec(shape, lambda i: (0,) * len(shape))


def _node_enc_body(x_ref, w1, b1, w2, b2, w1sd, hn_ref, pt_ref):
    h = jnp.maximum(_dot(x_ref[...], w1[...]) + b1[...], 0.0)
    hn = _dot(h, w2[...]) + b2[...]
    hn_ref[...] = hn
    pt_ref[...] = _dot(hn, w1sd[...])


def _edge_step1_body(ea_ref, gs_ref, gd_ref, ew1, eb1, ew2, eb2,
                     w1e, b1, w2, b2, out_ref):
    eh = jnp.maximum(_dot(ea_ref[...], ew1[...]) + eb1[...], 0.0)
    he = _dot(eh, ew2[...]) + eb2[...]
    z = jnp.maximum(
        _dot(he, w1e[...]) + gs_ref[:, 0:_H] + gd_ref[:, _H:2 * _H]
        + b1[...], 0.0)
    en = _dot(z, w2[...]) + b2[...]
    out_ref[...] = jnp.concatenate([en, he + en], axis=1)


def _edge_step2_body(eh_ref, gs_ref, gd_ref, w1e, b1, w2, b2, out_ref):
    z = jnp.maximum(
        _dot(eh_ref[:, _H:2 * _H], w1e[...]) + gs_ref[:, 0:_H]
        + gd_ref[:, _H:2 * _H] + b1[...], 0.0)
    en = _dot(z, w2[...]) + b2[...]
    out_ref[...] = jnp.concatenate([en, jnp.zeros_like(en)], axis=1)


def _node_upd_body(hn_ref, parts_ref, w1h, w1a, b1, w2, b2, w1sd,
                   hn2_ref, pt_ref):
    agg = parts_ref[0, :, 0:_H] + parts_ref[1, :, 0:_H]
    h = jnp.maximum(
        _dot(hn_ref[...], w1h[...]) + _dot(agg, w1a[...]) + b1[...], 0.0)
    hn2 = hn_ref[...] + _dot(h, w2[...]) + b2[...]
    hn2_ref[...] = hn2
    pt_ref[...] = _dot(hn2, w1sd[...])


def _node_final_body(hn_ref, parts_ref, w1h, w1a, b1, w2, b2, dw1, db1,
                     dw2, db2, out_ref):
    agg = parts_ref[0, :, 0:_H] + parts_ref[1, :, 0:_H]
    h = jnp.maximum(
        _dot(hn_ref[...], w1h[...]) + _dot(agg, w1a[...]) + b1[...], 0.0)
    hn2 = hn_ref[...] + _dot(h, w2[...]) + b2[...]
    d = jnp.maximum(_dot(hn2, dw1[...]) + db1[...], 0.0)
    out_ref[...] = _dot(d, dw2[...]) + db2[...]


def kernel(x, edge_index, edge_attr, params):
    f32 = jnp.float32
    src = edge_index[0].astype(jnp.int32)
    dst = edge_index[1].astype(jnp.int32)
    src2 = src.reshape(_NW, _NCH, _C)
    dst2 = dst.reshape(_NW, _NCH, _C)
    zeros_tile = jnp.zeros((_NWB, 2 * _H), f32)

    pn, pe = params["enc_node"], params["enc_edge"]
    dec = params["dec_node"]

    def row(b):
        return b.reshape(1, -1)

    ew = [params["proc"][i]["edge"] for i in range(2)]
    nw = [params["proc"][i]["node"] for i in range(2)]
    w1sd = [jnp.concatenate([ew[i]["w1"][64:128], ew[i]["w1"][128:192]],
                            axis=1) for i in range(2)]

    grid_n = _N // _BN
    grid_e = _E // _BE
    bspec_n = pl.BlockSpec((_BN, _H), lambda i: (i, 0))
    bspec_n2 = pl.BlockSpec((_BN, 2 * _H), lambda i: (i, 0))
    bspec_e2 = pl.BlockSpec((_BE, 2 * _H), lambda i: (i, 0))
    w_spec = _full((_H, _H))
    wsd_spec = _full((_H, 2 * _H))
    b_spec = _full((1, _H))
    parts_spec = pl.BlockSpec((_NC, _BN, 2 * _H), lambda i: (0, i, 0))

    hn, pt1 = pl.pallas_call(
        _node_enc_body,
        grid=(grid_n,),
        in_specs=[pl.BlockSpec((_BN, 128), lambda i: (i, 0)),
                  _full((128, _H)), b_spec, w_spec, b_spec, wsd_spec],
        out_specs=[bspec_n, bspec_n2],
        out_shape=[jax.ShapeDtypeStruct((_N, _H), f32),
                   jax.ShapeDtypeStruct((_N, 2 * _H), f32)],
    )(x, pn["w1"], row(pn["b1"]), pn["w2"], row(pn["b2"]), w1sd[0])

    for step in range(2):
        gs, gd = _sc_gather(pt1, src2, dst2)
        if step == 0:
            enhen = pl.pallas_call(
                _edge_step1_body,
                grid=(grid_e,),
                in_specs=[pl.BlockSpec((_BE, 16), lambda i: (i, 0)),
                          bspec_e2, bspec_e2,
                          _full((16, _H)), b_spec, w_spec, b_spec,
                          w_spec, b_spec, w_spec, b_spec],
                out_specs=bspec_e2,
                out_shape=jax.ShapeDtypeStruct((_E, 2 * _H), f32),
            )(edge_attr, gs, gd, pe["w1"], row(pe["b1"]), pe["w2"],
              row(pe["b2"]), ew[0]["w1"][0:64], row(ew[0]["b1"]),
              ew[0]["w2"], row(ew[0]["b2"]))
        else:
            enhen = pl.pallas_call(
                _edge_step2_body,
                grid=(grid_e,),
                in_specs=[bspec_e2, bspec_e2, bspec_e2,
                          w_spec, b_spec, w_spec, b_spec],
                out_specs=bspec_e2,
                out_shape=jax.ShapeDtypeStruct((_E, 2 * _H), f32),
            )(prev_enhen, gs, gd, ew[1]["w1"][0:64], row(ew[1]["b1"]),
              ew[1]["w2"], row(ew[1]["b2"]))

        parts = _sc_scatter(enhen.reshape(_E // _C, _C, 2 * _H), dst2,
                            zeros_tile)
        prev_enhen = enhen

        if step == 0:
            hn, pt1 = pl.pallas_call(
                _node_upd_body,
                grid=(grid_n,),
                in_specs=[bspec_n, parts_spec, w_spec, w_spec, b_spec,
                          w_spec, b_spec, wsd_spec],
                out_specs=[bspec_n, bspec_n2],
                out_shape=[jax.ShapeDtypeStruct((_N, _H), f32),
                           jax.ShapeDtypeStruct((_N, 2 * _H), f32)],
            )(hn, parts, nw[0]["w1"][0:64], nw[0]["w1"][64:128],
              row(nw[0]["b1"]), nw[0]["w2"], row(nw[0]["b2"]), w1sd[1])
        else:
            out = pl.pallas_call(
                _node_final_body,
                grid=(grid_n,),
                in_specs=[bspec_n, parts_spec, w_spec, w_spec, b_spec,
                          w_spec, b_spec, w_spec, b_spec,
                          _full((_H, 4)), _full((1, 4))],
                out_specs=pl.BlockSpec((_BN, 4), lambda i: (i, 0)),
                out_shape=jax.ShapeDtypeStruct((_N, 4), f32),
            )(hn, parts, nw[1]["w1"][0:64], nw[1]["w1"][64:128],
              row(nw[1]["b1"]), nw[1]["w2"], row(nw[1]["b2"]),
              dec["w1"], row(dec["b1"]), dec["w2"], row(dec["b2"]))
    return out
